# baseline (speedup 1.0000x reference)
import functools

import jax
import jax.numpy as jnp
from jax import lax
from jax.experimental import pallas as pl
from jax.experimental.pallas import tpu as pltpu
from jax.experimental.pallas import tpu_sc as plsc

_N = 10000
_NP = 10112
_RPT = _NP // 16
_BROWS = 112
_NTILES = 16
_WB = 15
_NW = 6
_K = 3


def _leaky(y):
    return jnp.where(y >= 0, y, y * 0.2)


def _deg_sc(dst_slabs, ones_h, zeros_h):
    mesh = plsc.VectorSubcoreMesh(
        core_axis_name="c", subcore_axis_name="s", num_cores=2, num_subcores=16
    )

    @functools.partial(
        pl.kernel,
        out_type=jax.ShapeDtypeStruct((2, _NP, 128), jnp.float32),
        mesh=mesh,
        scratch_types=[
            pltpu.VMEM((_WB, _BROWS), jnp.int32),
            pltpu.VMEM((_BROWS, 128), jnp.float32),
            pltpu.VMEM_SHARED((_NP, 128), jnp.float32),
        ],
    )
    def k(dst_hbm, ones_hbm, z_hbm, out_hbm, dst_v, ones_v, acc):
        core = lax.axis_index("c")
        sid = lax.axis_index("s")
        row0 = sid * _RPT
        pltpu.sync_copy(ones_hbm, ones_v)
        pltpu.sync_copy(z_hbm, acc.at[pl.ds(row0, _RPT)])
        plsc.subcore_barrier()
        w_lo = core * (_NW // 2)

        def window_body(w, _):
            pltpu.sync_copy(dst_hbm.at[sid, w], dst_v)
            for j in range(_WB):
                pltpu.sync_copy(ones_v, acc.at[dst_v.at[j]], add=True)
            return 0

        lax.fori_loop(w_lo, w_lo + _NW // 2, window_body, 0)
        plsc.subcore_barrier()
        pltpu.sync_copy(acc.at[pl.ds(row0, _RPT)],
                        out_hbm.at[core, pl.ds(row0, _RPT)])

    return k(dst_slabs, ones_h, zeros_h)


def _segsum_sc(h3, src_slabs, dst_slabs, zeros_h, *, C, nch, split):
    mesh = plsc.VectorSubcoreMesh(
        core_axis_name="c", subcore_axis_name="s", num_cores=2, num_subcores=16
    )

    @functools.partial(
        pl.kernel,
        out_type=jax.ShapeDtypeStruct((2 if split else nch, _NP, C), jnp.float32),
        mesh=mesh,
        scratch_types=(
            [pltpu.VMEM((_WB, _BROWS), jnp.int32)] * 2
            + [pltpu.VMEM((_BROWS, C), jnp.float32)] * _K
            + [pltpu.VMEM_SHARED((_NP, C), jnp.float32)]
            + [pltpu.SemaphoreType.DMA] * (2 * _K)
        ),
    )
    def k(h_hbm, src_hbm, dst_hbm, z_hbm, out_hbm, src_v, dst_v, *rest):
        bufs = rest[:_K]
        acc = rest[_K]
        gsems = rest[_K + 1:2 * _K + 1]
        ssems = rest[2 * _K + 1:]
        core = lax.axis_index("c")
        sid = lax.axis_index("s")
        row0 = sid * _RPT

        def run_chunk(c, out_slot, w_lo, w_hi):
            hc = h_hbm.at[c]
            pltpu.sync_copy(z_hbm, acc.at[pl.ds(row0, _RPT)])
            plsc.subcore_barrier()

            def window_body(w, _):
                pltpu.sync_copy(src_hbm.at[sid, w], src_v)
                pltpu.sync_copy(dst_hbm.at[sid, w], dst_v)
                for p in range(_K - 1):
                    pltpu.async_copy(hc.at[src_v.at[p]], bufs[p], gsems[p], priority=1)
                for j in range(_WB):
                    s = j % _K
                    pltpu.make_async_copy(hc.at[src_v.at[j]], bufs[s], gsems[s]).wait()
                    pltpu.async_copy(bufs[s], acc.at[dst_v.at[j]], ssems[s], add=True)
                    if j + _K - 1 < _WB:
                        s2 = (j + _K - 1) % _K
                        if j >= 1:
                            pltpu.make_async_copy(
                                bufs[s2], acc.at[dst_v.at[j - 1]], ssems[s2]
                            ).wait()
                        pltpu.async_copy(hc.at[src_v.at[j + _K - 1]], bufs[s2],
                                         gsems[s2], priority=1)
                for j in range(_WB - _K, _WB):
                    s = j % _K
                    pltpu.make_async_copy(bufs[s], acc.at[dst_v.at[j]], ssems[s]).wait()
                return 0

            lax.fori_loop(w_lo, w_hi, window_body, 0)
            plsc.subcore_barrier()
            pltpu.sync_copy(acc.at[pl.ds(row0, _RPT)],
                            out_hbm.at[out_slot, pl.ds(row0, _RPT)])

        if split:
            w_lo = core * (_NW // 2)
            run_chunk(0, core, w_lo, w_lo + _NW // 2)
        else:
            def chunk_body(c, _):
                @pl.when(core == lax.rem(c, 2))
                def _():
                    run_chunk(c, c, 0, _NW)
                return 0

            lax.fori_loop(0, nch, chunk_body, 0)

    return k(h3, src_slabs, dst_slabs, zeros_h)


_RB = 2000


def _dinv_tc(deg):
    def body(deg_ref, out_ref):
        out_ref[...] = lax.rsqrt(deg_ref[0, :, :1] + deg_ref[1, :, :1] + 1.0)

    return pl.pallas_call(
        body,
        grid=(_N // _RB,),
        in_specs=[pl.BlockSpec((2, _RB, 128), lambda r: (0, r, 0))],
        out_specs=pl.BlockSpec((_RB, 1), lambda r: (r, 0)),
        out_shape=jax.ShapeDtypeStruct((_N, 1), jnp.float32),
    )(deg)


def _scale_tc(x, dinv):
    d = x.shape[1]

    def body(x_ref, dinv_ref, out_ref):
        out_ref[0] = x_ref[...] * dinv_ref[...]

    return pl.pallas_call(
        body,
        grid=(_N // _RB,),
        in_specs=[
            pl.BlockSpec((_RB, d), lambda r: (r, 0)),
            pl.BlockSpec((_RB, 1), lambda r: (r, 0)),
        ],
        out_specs=pl.BlockSpec((1, _RB, d), lambda r: (0, r, 0)),
        out_shape=jax.ShapeDtypeStruct((1, _N, d), jnp.float32),
    )(x, dinv)


def _matmul_tc(x, w, b, dinv, *, bias_act, post_dinv, chunk_out):
    n, din = x.shape
    dout = w.shape[1]
    cb = min(512, dout)
    nc = dout // cb
    cbc = cb // 128

    def body(x_ref, w_ref, b_ref, dinv_ref, out_ref):
        y = jnp.dot(x_ref[...], w_ref[...], preferred_element_type=jnp.float32)
        if bias_act:
            y = _leaky(y + b_ref[...])
        if post_dinv:
            y = y * dinv_ref[...]
        if chunk_out:
            for i in range(cbc):
                out_ref[i] = y[:, i * 128:(i + 1) * 128]
        else:
            out_ref[...] = y

    if chunk_out:
        out_spec = pl.BlockSpec((cbc, _RB, 128), lambda r, c: (c, r, 0))
        out_shape = jax.ShapeDtypeStruct((dout // 128, n, 128), jnp.float32)
    else:
        out_spec = pl.BlockSpec((_RB, cb), lambda r, c: (r, c))
        out_shape = jax.ShapeDtypeStruct((n, dout), jnp.float32)

    return pl.pallas_call(
        body,
        grid=(n // _RB, nc),
        in_specs=[
            pl.BlockSpec((_RB, din), lambda r, c: (r, 0)),
            pl.BlockSpec((din, cb), lambda r, c: (0, c)),
            pl.BlockSpec((1, cb), lambda r, c: (0, c)),
            pl.BlockSpec((_RB, 1), lambda r, c: (r, 0)),
        ],
        out_specs=out_spec,
        out_shape=out_shape,
    )(x, w, b.reshape(1, dout), dinv)


def _matmul_fused_tc(t3, xs3, dinv, w3, b, *, tsplit, post_dinv, chunk_out):
    nch = xs3.shape[0]
    dout = w3.shape[2]
    cb = min(512, dout)
    tb = 2 if tsplit else nch
    rb = 1000
    cbc = cb // 128

    def body(t_ref, xs_ref, dinv_ref, w_ref, b_ref, out_ref):
        y = None
        for c in range(nch):
            t_c = t_ref[0] + t_ref[1] if tsplit else t_ref[c]
            zc = (t_c + xs_ref[c]) * dinv_ref[...]
            d = jnp.dot(zc, w_ref[c], preferred_element_type=jnp.float32)
            y = d if y is None else y + d
        y = _leaky(y + b_ref[...])
        if post_dinv:
            y = y * dinv_ref[...]
        if chunk_out:
            for i in range(cbc):
                out_ref[i] = y[:, i * 128:(i + 1) * 128]
        else:
            out_ref[...] = y

    if chunk_out:
        out_spec = pl.BlockSpec((cbc, rb, 128), lambda r, c: (c, r, 0))
        out_shape = jax.ShapeDtypeStruct((dout // 128, _N, 128), jnp.float32)
    else:
        out_spec = pl.BlockSpec((rb, cb), lambda r, c: (r, c))
        out_shape = jax.ShapeDtypeStruct((_N, dout), jnp.float32)

    return pl.pallas_call(
        body,
        grid=(_N // rb, dout // cb),
        in_specs=[
            pl.BlockSpec((tb, rb, 128), lambda r, c: (0, r, 0)),
            pl.BlockSpec((nch, rb, 128), lambda r, c: (0, r, 0)),
            pl.BlockSpec((rb, 1), lambda r, c: (r, 0)),
            pl.BlockSpec((nch, 128, cb), lambda r, c: (0, 0, c)),
            pl.BlockSpec((1, cb), lambda r, c: (0, c)),
        ],
        out_specs=out_spec,
        out_shape=out_shape,
    )(t3, xs3, dinv, w3, b.reshape(1, dout))


def _combine_tc(t3, xs3, dinv, b, *, bias_act, post_dinv, chunk_out, tsplit=False):
    nch, _, C = xs3.shape
    d = nch * C
    tb = 2 if tsplit else 1

    def body(t_ref, xs_ref, dinv_ref, b_ref, out_ref):
        t = t_ref[0] + t_ref[1] if tsplit else t_ref[0]
        y = (t + xs_ref[0]) * dinv_ref[...]
        if bias_act:
            y = _leaky(y + b_ref[...])
        if post_dinv:
            y = y * dinv_ref[...]
        if chunk_out:
            out_ref[0] = y
        else:
            out_ref[...] = y

    if chunk_out:
        out_spec = pl.BlockSpec((1, _RB, C), lambda c, r: (c, r, 0))
        out_shape = jax.ShapeDtypeStruct((nch, _N, C), jnp.float32)
    else:
        out_spec = pl.BlockSpec((_RB, C), lambda c, r: (r, c))
        out_shape = jax.ShapeDtypeStruct((_N, d), jnp.float32)

    return pl.pallas_call(
        body,
        grid=(nch, _N // _RB),
        in_specs=[
            pl.BlockSpec((tb, _RB, C), lambda c, r: (0 if tsplit else c, r, 0)),
            pl.BlockSpec((1, _RB, C), lambda c, r: (c, r, 0)),
            pl.BlockSpec((_RB, 1), lambda c, r: (r, 0)),
            pl.BlockSpec((1, C), lambda c, r: (0, c)),
        ],
        out_specs=out_spec,
        out_shape=out_shape,
    )(t3, xs3, dinv, b.reshape(1, d))


def _final_tc(t3, h3, dinv, b, nclass):
    C = h3.shape[2]

    def body(t_ref, h_ref, dinv_ref, b_ref, out_ref):
        y = _leaky((t_ref[0] + t_ref[1] + h_ref[0]) * dinv_ref[...] + b_ref[...])
        y = y[:, :nclass]
        m = jnp.max(y, axis=1, keepdims=True)
        out_ref[...] = y - m - jnp.log(jnp.sum(jnp.exp(y - m), axis=1, keepdims=True))

    return pl.pallas_call(
        body,
        grid=(_N // _RB,),
        in_specs=[
            pl.BlockSpec((2, _RB, C), lambda r: (0, r, 0)),
            pl.BlockSpec((1, _RB, C), lambda r: (0, r, 0)),
            pl.BlockSpec((_RB, 1), lambda r: (r, 0)),
            pl.BlockSpec((1, C), lambda r: (0, 0)),
        ],
        out_specs=pl.BlockSpec((_RB, nclass), lambda r: (r, 0)),
        out_shape=jax.ShapeDtypeStruct((_N, nclass), jnp.float32),
    )(t3, h3, dinv, b.reshape(1, C))


def kernel(x, edge_index, W1, b1, W2, b2, W3, b3, W4, b4, W5, b5, W6, b6):
    E = edge_index.shape[1]
    nb = _WB * _NW
    Ep = nb * _NTILES * _BROWS
    assert E <= Ep

    src = edge_index[0].astype(jnp.int32)
    dst = edge_index[1].astype(jnp.int32)
    pad = Ep - E
    src_slabs = jnp.concatenate([src, jnp.zeros((pad,), jnp.int32)]).reshape(
        _NTILES, _NW, _WB, _BROWS
    )
    dst_slabs = jnp.concatenate([dst, jnp.full((pad,), _N, jnp.int32)]).reshape(
        _NTILES, _NW, _WB, _BROWS
    )

    zeros_h = jnp.zeros((_RPT, 128), jnp.float32)

    def segsum(h3, nch, split=False):
        return _segsum_sc(h3, src_slabs, dst_slabs, zeros_h,
                          C=128, nch=nch, split=split)

    deg = _deg_sc(dst_slabs, jnp.ones((_BROWS, 128), jnp.float32), zeros_h)
    dinv = _dinv_tc(deg)

    x1s = _scale_tc(x, dinv)
    t1 = segsum(x1s, 1, split=True)
    x2s = _matmul_fused_tc(t1, x1s, dinv, W1.reshape(1, 128, 2048), b1,
                           tsplit=True, post_dinv=True, chunk_out=True)

    t2 = segsum(x2s, 16)
    x3 = _matmul_fused_tc(t2, x2s, dinv, W2.reshape(16, 128, 2048), b2,
                          tsplit=False, post_dinv=False, chunk_out=False)

    h3 = _matmul_tc(x3, W3, b3, dinv, bias_act=False, post_dinv=True, chunk_out=True)
    t3 = segsum(h3, 8)
    x4s = _combine_tc(t3, h3, dinv, b3, bias_act=True, post_dinv=True, chunk_out=True)

    t4 = segsum(x4s, 8)
    x5 = _matmul_fused_tc(t4, x4s, dinv, W4.reshape(8, 128, 1024), b4,
                          tsplit=False, post_dinv=False, chunk_out=False)

    h5 = _matmul_tc(x5, W5, b5, dinv, bias_act=False, post_dinv=True, chunk_out=True)
    t5 = segsum(h5, 4)
    x6 = _combine_tc(t5, h5, dinv, b5, bias_act=True, post_dinv=False, chunk_out=False)

    nclass = W6.shape[1]
    W6p = jnp.pad(W6, ((0, 0), (0, 128 - nclass)))
    b6p = jnp.pad(b6, (0, 128 - nclass))
    h6 = _matmul_tc(x6, W6p, b6p, dinv, bias_act=False, post_dinv=True, chunk_out=True)
    t6 = segsum(h6, 1, split=True)
    return _final_tc(t6, h6, dinv, b6p, nclass)


# speedup vs baseline: 1.0015x; 1.0015x over previous
import functools

import jax
import jax.numpy as jnp
from jax import lax
from jax.experimental import pallas as pl
from jax.experimental.pallas import tpu as pltpu
from jax.experimental.pallas import tpu_sc as plsc

_N = 10000
_NP = 10112
_RPT = _NP // 16
_BROWS = 112
_NTILES = 16
_WB = 15
_NW = 6
_K = 3


def _leaky(y):
    return jnp.where(y >= 0, y, y * 0.2)


def _deg_sc(dst_slabs, ones_h, zeros_h):
    mesh = plsc.VectorSubcoreMesh(
        core_axis_name="c", subcore_axis_name="s", num_cores=2, num_subcores=16
    )

    @functools.partial(
        pl.kernel,
        out_type=jax.ShapeDtypeStruct((2, _NP, 128), jnp.float32),
        mesh=mesh,
        scratch_types=[
            pltpu.VMEM((_WB, _BROWS), jnp.int32),
            pltpu.VMEM((_BROWS, 128), jnp.float32),
            pltpu.VMEM_SHARED((_NP, 128), jnp.float32),
        ],
    )
    def k(dst_hbm, ones_hbm, z_hbm, out_hbm, dst_v, ones_v, acc):
        core = lax.axis_index("c")
        sid = lax.axis_index("s")
        row0 = sid * _RPT
        pltpu.sync_copy(ones_hbm, ones_v)
        pltpu.sync_copy(z_hbm, acc.at[pl.ds(row0, _RPT)])
        plsc.subcore_barrier()
        w_lo = core * (_NW // 2)

        def window_body(w, _):
            pltpu.sync_copy(dst_hbm.at[sid, w], dst_v)
            for j in range(_WB):
                pltpu.sync_copy(ones_v, acc.at[dst_v.at[j]], add=True)
            return 0

        lax.fori_loop(w_lo, w_lo + _NW // 2, window_body, 0)
        plsc.subcore_barrier()
        pltpu.sync_copy(acc.at[pl.ds(row0, _RPT)],
                        out_hbm.at[core, pl.ds(row0, _RPT)])

    return k(dst_slabs, ones_h, zeros_h)


def _segsum_sc(h3, src_slabs, dst_slabs, zeros_h, *, C, nch, split):
    mesh = plsc.VectorSubcoreMesh(
        core_axis_name="c", subcore_axis_name="s", num_cores=2, num_subcores=16
    )

    @functools.partial(
        pl.kernel,
        out_type=jax.ShapeDtypeStruct((2 if split else nch, _NP, C), jnp.float32),
        mesh=mesh,
        scratch_types=(
            [pltpu.VMEM((_WB, _BROWS), jnp.int32)] * 2
            + [pltpu.VMEM((_BROWS, C), jnp.float32)] * _K
            + [pltpu.VMEM_SHARED((_NP, C), jnp.float32)]
            + [pltpu.SemaphoreType.DMA] * (2 * _K)
        ),
    )
    def k(h_hbm, src_hbm, dst_hbm, z_hbm, out_hbm, src_v, dst_v, *rest):
        bufs = rest[:_K]
        acc = rest[_K]
        gsems = rest[_K + 1:2 * _K + 1]
        ssems = rest[2 * _K + 1:]
        core = lax.axis_index("c")
        sid = lax.axis_index("s")
        row0 = sid * _RPT

        def run_chunk(c, out_slot, w_lo, w_hi):
            hc = h_hbm.at[c]
            pltpu.sync_copy(z_hbm, acc.at[pl.ds(row0, _RPT)])
            plsc.subcore_barrier()

            def window_body(w, _):
                pltpu.sync_copy(src_hbm.at[sid, w], src_v)
                pltpu.sync_copy(dst_hbm.at[sid, w], dst_v)
                for p in range(_K - 1):
                    pltpu.async_copy(hc.at[src_v.at[p]], bufs[p], gsems[p])
                for j in range(_WB):
                    s = j % _K
                    pltpu.make_async_copy(hc.at[src_v.at[j]], bufs[s], gsems[s]).wait()
                    pltpu.async_copy(bufs[s], acc.at[dst_v.at[j]], ssems[s], add=True)
                    if j + _K - 1 < _WB:
                        s2 = (j + _K - 1) % _K
                        if j >= 1:
                            pltpu.make_async_copy(
                                bufs[s2], acc.at[dst_v.at[j - 1]], ssems[s2]
                            ).wait()
                        pltpu.async_copy(hc.at[src_v.at[j + _K - 1]], bufs[s2],
                                         gsems[s2])
                for j in range(_WB - _K, _WB):
                    s = j % _K
                    pltpu.make_async_copy(bufs[s], acc.at[dst_v.at[j]], ssems[s]).wait()
                return 0

            lax.fori_loop(w_lo, w_hi, window_body, 0)
            plsc.subcore_barrier()
            pltpu.sync_copy(acc.at[pl.ds(row0, _RPT)],
                            out_hbm.at[out_slot, pl.ds(row0, _RPT)])

        if split:
            w_lo = core * (_NW // 2)
            run_chunk(0, core, w_lo, w_lo + _NW // 2)
        else:
            def chunk_body(c, _):
                @pl.when(core == lax.rem(c, 2))
                def _():
                    run_chunk(c, c, 0, _NW)
                return 0

            lax.fori_loop(0, nch, chunk_body, 0)

    return k(h3, src_slabs, dst_slabs, zeros_h)


_RB = 2000


def _dinv_tc(deg):
    def body(deg_ref, out_ref):
        out_ref[...] = lax.rsqrt(deg_ref[0, :, :1] + deg_ref[1, :, :1] + 1.0)

    return pl.pallas_call(
        body,
        grid=(_N // _RB,),
        in_specs=[pl.BlockSpec((2, _RB, 128), lambda r: (0, r, 0))],
        out_specs=pl.BlockSpec((_RB, 1), lambda r: (r, 0)),
        out_shape=jax.ShapeDtypeStruct((_N, 1), jnp.float32),
    )(deg)


def _scale_tc(x, dinv):
    d = x.shape[1]

    def body(x_ref, dinv_ref, out_ref):
        out_ref[0] = x_ref[...] * dinv_ref[...]

    return pl.pallas_call(
        body,
        grid=(_N // _RB,),
        in_specs=[
            pl.BlockSpec((_RB, d), lambda r: (r, 0)),
            pl.BlockSpec((_RB, 1), lambda r: (r, 0)),
        ],
        out_specs=pl.BlockSpec((1, _RB, d), lambda r: (0, r, 0)),
        out_shape=jax.ShapeDtypeStruct((1, _N, d), jnp.float32),
    )(x, dinv)


def _matmul_tc(x, w, b, dinv, *, bias_act, post_dinv, chunk_out):
    n, din = x.shape
    dout = w.shape[1]
    cb = min(512, dout)
    nc = dout // cb
    cbc = cb // 128

    def body(x_ref, w_ref, b_ref, dinv_ref, out_ref):
        y = jnp.dot(x_ref[...], w_ref[...], preferred_element_type=jnp.float32)
        if bias_act:
            y = _leaky(y + b_ref[...])
        if post_dinv:
            y = y * dinv_ref[...]
        if chunk_out:
            for i in range(cbc):
                out_ref[i] = y[:, i * 128:(i + 1) * 128]
        else:
            out_ref[...] = y

    if chunk_out:
        out_spec = pl.BlockSpec((cbc, _RB, 128), lambda r, c: (c, r, 0))
        out_shape = jax.ShapeDtypeStruct((dout // 128, n, 128), jnp.float32)
    else:
        out_spec = pl.BlockSpec((_RB, cb), lambda r, c: (r, c))
        out_shape = jax.ShapeDtypeStruct((n, dout), jnp.float32)

    return pl.pallas_call(
        body,
        grid=(n // _RB, nc),
        in_specs=[
            pl.BlockSpec((_RB, din), lambda r, c: (r, 0)),
            pl.BlockSpec((din, cb), lambda r, c: (0, c)),
            pl.BlockSpec((1, cb), lambda r, c: (0, c)),
            pl.BlockSpec((_RB, 1), lambda r, c: (r, 0)),
        ],
        out_specs=out_spec,
        out_shape=out_shape,
    )(x, w, b.reshape(1, dout), dinv)


def _matmul_fused_tc(t3, xs3, dinv, w3, b, *, tsplit, post_dinv, chunk_out):
    nch = xs3.shape[0]
    dout = w3.shape[2]
    cb = min(512, dout)
    tb = 2 if tsplit else nch
    rb = 1000
    cbc = cb // 128

    def body(t_ref, xs_ref, dinv_ref, w_ref, b_ref, out_ref):
        y = None
        for c in range(nch):
            t_c = t_ref[0] + t_ref[1] if tsplit else t_ref[c]
            zc = (t_c + xs_ref[c]) * dinv_ref[...]
            d = jnp.dot(zc, w_ref[c], preferred_element_type=jnp.float32)
            y = d if y is None else y + d
        y = _leaky(y + b_ref[...])
        if post_dinv:
            y = y * dinv_ref[...]
        if chunk_out:
            for i in range(cbc):
                out_ref[i] = y[:, i * 128:(i + 1) * 128]
        else:
            out_ref[...] = y

    if chunk_out:
        out_spec = pl.BlockSpec((cbc, rb, 128), lambda r, c: (c, r, 0))
        out_shape = jax.ShapeDtypeStruct((dout // 128, _N, 128), jnp.float32)
    else:
        out_spec = pl.BlockSpec((rb, cb), lambda r, c: (r, c))
        out_shape = jax.ShapeDtypeStruct((_N, dout), jnp.float32)

    return pl.pallas_call(
        body,
        grid=(_N // rb, dout // cb),
        in_specs=[
            pl.BlockSpec((tb, rb, 128), lambda r, c: (0, r, 0)),
            pl.BlockSpec((nch, rb, 128), lambda r, c: (0, r, 0)),
            pl.BlockSpec((rb, 1), lambda r, c: (r, 0)),
            pl.BlockSpec((nch, 128, cb), lambda r, c: (0, 0, c)),
            pl.BlockSpec((1, cb), lambda r, c: (0, c)),
        ],
        out_specs=out_spec,
        out_shape=out_shape,
    )(t3, xs3, dinv, w3, b.reshape(1, dout))


def _combine_tc(t3, xs3, dinv, b, *, bias_act, post_dinv, chunk_out, tsplit=False):
    nch, _, C = xs3.shape
    d = nch * C
    tb = 2 if tsplit else 1

    def body(t_ref, xs_ref, dinv_ref, b_ref, out_ref):
        t = t_ref[0] + t_ref[1] if tsplit else t_ref[0]
        y = (t + xs_ref[0]) * dinv_ref[...]
        if bias_act:
            y = _leaky(y + b_ref[...])
        if post_dinv:
            y = y * dinv_ref[...]
        if chunk_out:
            out_ref[0] = y
        else:
            out_ref[...] = y

    if chunk_out:
        out_spec = pl.BlockSpec((1, _RB, C), lambda c, r: (c, r, 0))
        out_shape = jax.ShapeDtypeStruct((nch, _N, C), jnp.float32)
    else:
        out_spec = pl.BlockSpec((_RB, C), lambda c, r: (r, c))
        out_shape = jax.ShapeDtypeStruct((_N, d), jnp.float32)

    return pl.pallas_call(
        body,
        grid=(nch, _N // _RB),
        in_specs=[
            pl.BlockSpec((tb, _RB, C), lambda c, r: (0 if tsplit else c, r, 0)),
            pl.BlockSpec((1, _RB, C), lambda c, r: (c, r, 0)),
            pl.BlockSpec((_RB, 1), lambda c, r: (r, 0)),
            pl.BlockSpec((1, C), lambda c, r: (0, c)),
        ],
        out_specs=out_spec,
        out_shape=out_shape,
    )(t3, xs3, dinv, b.reshape(1, d))


def _final_tc(t3, h3, dinv, b, nclass):
    C = h3.shape[2]

    def body(t_ref, h_ref, dinv_ref, b_ref, out_ref):
        y = _leaky((t_ref[0] + t_ref[1] + h_ref[0]) * dinv_ref[...] + b_ref[...])
        y = y[:, :nclass]
        m = jnp.max(y, axis=1, keepdims=True)
        out_ref[...] = y - m - jnp.log(jnp.sum(jnp.exp(y - m), axis=1, keepdims=True))

    return pl.pallas_call(
        body,
        grid=(_N // _RB,),
        in_specs=[
            pl.BlockSpec((2, _RB, C), lambda r: (0, r, 0)),
            pl.BlockSpec((1, _RB, C), lambda r: (0, r, 0)),
            pl.BlockSpec((_RB, 1), lambda r: (r, 0)),
            pl.BlockSpec((1, C), lambda r: (0, 0)),
        ],
        out_specs=pl.BlockSpec((_RB, nclass), lambda r: (r, 0)),
        out_shape=jax.ShapeDtypeStruct((_N, nclass), jnp.float32),
    )(t3, h3, dinv, b.reshape(1, C))


def kernel(x, edge_index, W1, b1, W2, b2, W3, b3, W4, b4, W5, b5, W6, b6):
    E = edge_index.shape[1]
    nb = _WB * _NW
    Ep = nb * _NTILES * _BROWS
    assert E <= Ep

    src = edge_index[0].astype(jnp.int32)
    dst = edge_index[1].astype(jnp.int32)
    pad = Ep - E
    src_slabs = jnp.concatenate([src, jnp.zeros((pad,), jnp.int32)]).reshape(
        _NTILES, _NW, _WB, _BROWS
    )
    dst_slabs = jnp.concatenate([dst, jnp.full((pad,), _N, jnp.int32)]).reshape(
        _NTILES, _NW, _WB, _BROWS
    )

    zeros_h = jnp.zeros((_RPT, 128), jnp.float32)

    def segsum(h3, nch, split=False):
        return _segsum_sc(h3, src_slabs, dst_slabs, zeros_h,
                          C=128, nch=nch, split=split)

    deg = _deg_sc(dst_slabs, jnp.ones((_BROWS, 128), jnp.float32), zeros_h)
    dinv = _dinv_tc(deg)

    x1s = _scale_tc(x, dinv)
    t1 = segsum(x1s, 1, split=True)
    x2s = _matmul_fused_tc(t1, x1s, dinv, W1.reshape(1, 128, 2048), b1,
                           tsplit=True, post_dinv=True, chunk_out=True)

    t2 = segsum(x2s, 16)
    x3 = _matmul_fused_tc(t2, x2s, dinv, W2.reshape(16, 128, 2048), b2,
                          tsplit=False, post_dinv=False, chunk_out=False)

    h3 = _matmul_tc(x3, W3, b3, dinv, bias_act=False, post_dinv=True, chunk_out=True)
    t3 = segsum(h3, 8)
    x4s = _combine_tc(t3, h3, dinv, b3, bias_act=True, post_dinv=True, chunk_out=True)

    t4 = segsum(x4s, 8)
    x5 = _matmul_fused_tc(t4, x4s, dinv, W4.reshape(8, 128, 1024), b4,
                          tsplit=False, post_dinv=False, chunk_out=False)

    h5 = _matmul_tc(x5, W5, b5, dinv, bias_act=False, post_dinv=True, chunk_out=True)
    t5 = segsum(h5, 4)
    x6 = _combine_tc(t5, h5, dinv, b5, bias_act=True, post_dinv=False, chunk_out=False)

    nclass = W6.shape[1]
    W6p = jnp.pad(W6, ((0, 0), (0, 128 - nclass)))
    b6p = jnp.pad(b6, (0, 128 - nclass))
    h6 = _matmul_tc(x6, W6p, b6p, dinv, bias_act=False, post_dinv=True, chunk_out=True)
    t6 = segsum(h6, 1, split=True)
    return _final_tc(t6, h6, dinv, b6p, nclass)
